# baseline (speedup 1.0000x reference)
import functools

import jax
import jax.numpy as jnp
from jax import lax
from jax.experimental import pallas as pl
from jax.experimental.pallas import tpu as pltpu
from jax.experimental.pallas import tpu_sc as plsc

NUM_POSITIONS = 2048
D = 2048
B = 4 * 2048

NC, NS = 2, 16
NW = NC * NS
CHUNK = 8
NBUF = 3


def _sc_gather(table, idx_flat, n_rows):
    b_per_w = n_rows // NW
    nchunk = b_per_w // CHUNK
    mesh = plsc.VectorSubcoreMesh(core_axis_name="c", subcore_axis_name="s")

    @functools.partial(
        pl.kernel,
        out_type=jax.ShapeDtypeStruct((n_rows, D), jnp.float32),
        mesh=mesh,
        scratch_types=(
            [pltpu.VMEM((b_per_w,), jnp.int32)]
            + [pltpu.VMEM((CHUNK, D), jnp.float32) for _ in range(NBUF)]
            + [pltpu.VMEM_SHARED((NS, NBUF, CHUNK, D), jnp.float32)]
            + [pltpu.SemaphoreType.DMA for _ in range(3 * NBUF)]
        ),
    )
    def k(table_hbm, idx_hbm, out_hbm, idx_v, *rest):
        tbufs = rest[:NBUF]
        shared = rest[NBUF]
        gsem = rest[NBUF + 1:2 * NBUF + 1]
        xsem = rest[2 * NBUF + 1:3 * NBUF + 1]
        ssem = rest[3 * NBUF + 1:]

        sid = lax.axis_index("s")
        sbufs = [shared.at[sid, b] for b in range(NBUF)]
        wid = sid * NC + lax.axis_index("c")
        base = wid * b_per_w
        pltpu.sync_copy(idx_hbm.at[pl.ds(base, b_per_w)], idx_v)

        def issue_gather(c):
            b = c % NBUF
            return pltpu.async_copy(
                table_hbm.at[idx_v.at[pl.ds(c * CHUNK, CHUNK)]],
                tbufs[b],
                gsem[b],
            )

        gather_cp = [None] * NBUF
        store_cp = [None] * NBUF
        for c in range(min(NBUF, nchunk)):
            gather_cp[c] = issue_gather(c)
        for c in range(nchunk):
            b = c % NBUF
            gather_cp[b].wait()
            if store_cp[b] is not None:
                store_cp[b].wait()
                store_cp[b] = None
            pltpu.async_copy(tbufs[b], sbufs[b], xsem[b]).wait()
            store_cp[b] = pltpu.async_copy(
                sbufs[b],
                out_hbm.at[pl.ds(base + c * CHUNK, CHUNK)],
                ssem[b],
            )
            n = c + NBUF
            if n < nchunk:
                gather_cp[b] = issue_gather(n)
        for cp in store_cp:
            if cp is not None:
                cp.wait()

    return k(table, idx_flat)


def kernel(input_, W):
    idx_flat = input_.reshape(B).astype(jnp.int32)
    out = _sc_gather(W, idx_flat, B)
    return out.reshape(input_.shape[0], input_.shape[1], NUM_POSITIONS)


# speedup vs baseline: 1.0154x; 1.0154x over previous
import functools

import jax
import jax.numpy as jnp
from jax import lax
from jax.experimental import pallas as pl
from jax.experimental.pallas import tpu as pltpu
from jax.experimental.pallas import tpu_sc as plsc

NUM_POSITIONS = 2048
D = 2048
B = 4 * 2048

NC, NS = 2, 16
NW = NC * NS
CHUNK = 8
NBUF = 7


def _sc_gather(table, idx_flat, n_rows):
    b_per_w = n_rows // NW
    nchunk = b_per_w // CHUNK
    mesh = plsc.VectorSubcoreMesh(core_axis_name="c", subcore_axis_name="s")

    @functools.partial(
        pl.kernel,
        out_type=jax.ShapeDtypeStruct((n_rows, D), jnp.float32),
        mesh=mesh,
        scratch_types=(
            [pltpu.VMEM((b_per_w,), jnp.int32)]
            + [pltpu.VMEM((CHUNK, D), jnp.float32) for _ in range(NBUF)]
            + [pltpu.SemaphoreType.DMA for _ in range(2 * NBUF)]
        ),
    )
    def k(table_hbm, idx_hbm, out_hbm, idx_v, *rest):
        bufs = rest[:NBUF]
        gsem = rest[NBUF:2 * NBUF]
        osem = rest[2 * NBUF:]

        wid = lax.axis_index("s") * NC + lax.axis_index("c")
        base = wid * b_per_w
        pltpu.sync_copy(idx_hbm.at[pl.ds(base, b_per_w)], idx_v)

        def issue_gather(c):
            b = c % NBUF
            return pltpu.async_copy(
                table_hbm.at[idx_v.at[pl.ds(c * CHUNK, CHUNK)]],
                bufs[b],
                gsem[b],
            )

        gather_cp = [None] * NBUF
        out_cp = [None] * NBUF
        for c in range(min(NBUF, nchunk)):
            gather_cp[c] = issue_gather(c)
        for c in range(nchunk):
            b = c % NBUF
            gather_cp[b].wait()
            out_cp[b] = pltpu.async_copy(
                bufs[b],
                out_hbm.at[pl.ds(base + c * CHUNK, CHUNK)],
                osem[b],
            )
            n = c + NBUF
            if n < nchunk:
                out_cp[b].wait()
                gather_cp[b] = issue_gather(n)
                out_cp[b] = None
        for cp in out_cp:
            if cp is not None:
                cp.wait()

    return k(table, idx_flat)


def kernel(input_, W):
    idx_flat = input_.reshape(B).astype(jnp.int32)
    out = _sc_gather(W, idx_flat, B)
    return out.reshape(input_.shape[0], input_.shape[1], NUM_POSITIONS)
